# baseline (speedup 1.0000x reference)
import functools

import jax
import jax.numpy as jnp
from jax import lax
from jax.experimental import pallas as pl
from jax.experimental.pallas import tpu as pltpu
from jax.experimental.pallas import tpu_sc as plsc

D = 32
B_ROWS = 16384 * 26
NC = 2
NS = 16
NW = NC * NS
CHUNK = 128
ROWS_PER_W = B_ROWS // NW
NCHUNKS = ROWS_PER_W // CHUNK
NSLOT = 8
LOOK = 4

_mesh = plsc.VectorSubcoreMesh(core_axis_name="c", subcore_axis_name="s")


@functools.partial(
    pl.kernel,
    mesh=_mesh,
    out_type=jax.ShapeDtypeStruct((NW, NCHUNKS, CHUNK, D), jnp.float32),
    scratch_types=[
        pltpu.VMEM((NCHUNKS, CHUNK), jnp.int32),
        pltpu.VMEM((NSLOT, CHUNK, D), jnp.float32),
        [pltpu.SemaphoreType.DMA] * NSLOT,
        [pltpu.SemaphoreType.DMA] * NSLOT,
    ],
    compiler_params=pltpu.CompilerParams(use_tc_tiling_on_sc=False),
)
def _gather_kernel(idx_hbm, table_hbm, out_hbm, idx_v, rows_v, sems_g, sems_w):
    wid = lax.axis_index("s") * NC + lax.axis_index("c")
    pltpu.sync_copy(idx_hbm.at[wid], idx_v)

    def gather(j, b):
        return pltpu.make_async_copy(table_hbm.at[idx_v.at[j]], rows_v.at[b],
                                     sems_g[b])

    def write(j, b):
        return pltpu.make_async_copy(rows_v.at[b], out_hbm.at[wid, j],
                                     sems_w[b])


    for j in range(LOOK):
        gather(j, j).start()
    for j in range(NSLOT - LOOK):
        gather(j, j).wait()
        write(j, j).start()
        gather(j + LOOK, (j + LOOK) % NSLOT).start()

    base = NSLOT - LOOK
    steady = (NCHUNKS - LOOK - base) // NSLOT

    def group(g, carry):
        for b in range(NSLOT):
            j = g * NSLOT + b + base
            sj = (b + base) % NSLOT
            sw = b
            gather(j, sj).wait()
            write(j, sj).start()
            write(j - base, sw).wait()
            gather(j + LOOK, sw).start()
        return carry

    lax.fori_loop(0, steady, group, 0)

    for jj in range(steady * NSLOT + base, NCHUNKS):
        sj = jj % NSLOT
        gather(jj, sj).wait()
        write(jj, sj).start()
        write(jj - base, (jj - base) % NSLOT).wait()
    for jj in range(NCHUNKS - base, NCHUNKS):
        write(jj, jj % NSLOT).wait()


def kernel(token_ids, weight):
    ids = token_ids.astype(jnp.int32).reshape(NW, NCHUNKS, CHUNK)
    out = _gather_kernel(ids, weight)
    return out.reshape(token_ids.shape[0], token_ids.shape[1], D)
